# Initial kernel scaffold; baseline (speedup 1.0000x reference)
import jax
import jax.numpy as jnp
from jax.experimental import pallas as pl


def kernel(f_ha, pos_ha, edge_attr_ha, f_ca, pos_ca, edge_attr_ca, params, edge_index_ha, edge_index_ca):
    raise NotImplementedError("write your pallas kernel here")


# speedup vs baseline: 3.3222x; 3.3222x over previous
import functools
import math

import jax
import jax.numpy as jnp
from jax import lax
from jax.experimental import pallas as pl
from jax.experimental.pallas import tpu as pltpu
from jax.experimental.pallas import tpu_sc as plsc

F32 = jnp.float32
I32 = jnp.int32

_C = 32
_GAS = 16
_CK = 8
_HEADS = 4
_HD = 2
_BE = 2048


def _rup(x, m):
    return (x + m - 1) // m * m


def _sc_gather(table, idx):
    return jnp.take(table, idx, axis=0)


def _sc_scatter(data, idx, init, add):
    Ep = data.shape[0]
    rpt = Ep // 32
    core = (jnp.arange(Ep, dtype=I32) // rpt) % 2
    outs = []
    for c in range(2):
        sel = core == c
        idx_c = jnp.where(sel, idx, init.shape[0] - 1)
        if add:
            data_c = jnp.where(sel[:, None], data, 0.0)
            o = init.at[idx_c].add(data_c, mode="drop")
        else:
            o = init.at[idx_c].set(data, mode="drop")
        outs.append(o)
    return jnp.stack(outs)


def _dot(a, b):
    return jnp.dot(a, b, preferred_element_type=F32)


def _tc_geom(PS, PD, ea):
    Ep = PS.shape[0]

    def body(ps, pd, a, o):
        rv = pd[:, 0:3] - ps[:, 0:3]
        d = jnp.sqrt(jnp.sum(rv * rv, axis=1, keepdims=True) + 1e-9)
        rhat = rv / d
        z = jnp.zeros((_BE, 7), F32)
        o[...] = jnp.concatenate([a[...], d, rhat, z], axis=1)

    return pl.pallas_call(
        body,
        grid=(Ep // _BE,),
        in_specs=[
            pl.BlockSpec((_BE, 16), lambda i: (i, 0)),
            pl.BlockSpec((_BE, 16), lambda i: (i, 0)),
            pl.BlockSpec((_BE, 5), lambda i: (i, 0)),
        ],
        out_specs=pl.BlockSpec((_BE, 16), lambda i: (i, 0)),
        out_shape=jax.ShapeDtypeStruct((Ep, 16), F32),
    )(PS, PD, ea)


def _tc_init(fp, w1t, b1, w2t, b2):
    N8 = fp.shape[0]

    def body(f, w1, bb1, w2, bb2, o):
        x = _dot(f[...], w1[...]) + bb1[...]
        x = jnp.where(x > 0, x, jnp.exp(x) - 1.0)
        o[...] = _dot(x, w2[...]) + bb2[...]

    return pl.pallas_call(
        body, out_shape=jax.ShapeDtypeStruct((N8, _C), F32)
    )(fp, w1t, b1, w2t, b2)


def _tc_proj(h0, h1, q0t, q1t, k0t, k1t, v0t, v1t):
    N8 = h0.shape[0]

    def body(h0r, h1r, q0r, q1r, k0r, k1r, v0r, v1r, so, do):
        h0v = h0r[...]
        parts_s = [_dot(h0v, k0r[...])]
        for d in range(3):
            parts_s.append(_dot(h1r[d], k1r[...]))
        parts_s.append(_dot(h0v, v0r[...]))
        for d in range(3):
            parts_s.append(_dot(h1r[d], v1r[...]))
        so[...] = jnp.concatenate(parts_s, axis=1)
        parts_d = [_dot(h0v, q0r[...])]
        for d in range(3):
            parts_d.append(_dot(h1r[d], q1r[...]))
        do[...] = jnp.concatenate(parts_d, axis=1)

    return pl.pallas_call(
        body,
        out_shape=(
            jax.ShapeDtypeStruct((N8, 64), F32),
            jax.ShapeDtypeStruct((N8, 32), F32),
        ),
    )(h0, h1, q0t, q1t, k0t, k1t, v0t, v1t)


def _tc_edge1(K, Q, G, rw1t, rb1, rw2t, rb2, E_real):
    Ep = K.shape[0]
    denom = math.sqrt(_HD * 4.0)

    def body(kr, qr, gr, w1, bb1, w2, bb2, lo, vo):
        i = pl.program_id(0)
        g = gr[...]
        rf = g[:, 0:8]
        rhat = g[:, 6:9]
        r = jnp.maximum(_dot(rf, w1[...]) + bb1[...], 0.0)
        r = _dot(r, w2[...]) + bb2[...]
        w = [r[:, 8 * j:8 * j + 8] for j in range(8)]
        kk = kr[...]
        qq = qr[...]
        k0 = kk[:, 0:8]
        v0 = kk[:, 32:40]
        k1 = [kk[:, 8 + 8 * d:16 + 8 * d] for d in range(3)]
        v1 = [kk[:, 40 + 8 * d:48 + 8 * d] for d in range(3)]
        q0 = qq[:, 0:8]
        q1 = [qq[:, 8 + 8 * d:16 + 8 * d] for d in range(3)]
        k1r = sum(k1[d] * rhat[:, d:d + 1] for d in range(3))
        v1r = sum(v1[d] * rhat[:, d:d + 1] for d in range(3))
        key0 = w[0] * k0 + w[2] * k1r
        val0 = w[4] * v0 + w[6] * v1r
        acc = q0 * key0
        for d in range(3):
            key1d = w[3] * k1[d] + (w[1] * k0) * rhat[:, d:d + 1]
            acc = acc + q1[d] * key1d
        l4 = jnp.concatenate(
            [acc[:, 2 * h:2 * h + 1] + acc[:, 2 * h + 1:2 * h + 2]
             for h in range(4)], axis=1) / denom
        rows = i * _BE + lax.broadcasted_iota(I32, (_BE, 1), 0)
        mask = rows < E_real
        l16 = jnp.concatenate([l4, jnp.zeros((_BE, 12), F32)], axis=1)
        lo[...] = jnp.where(mask, l16, 0.0)
        val1s = [w[7] * v1[d] + (w[5] * v0) * rhat[:, d:d + 1] for d in range(3)]
        v32 = jnp.concatenate([val0] + val1s, axis=1)
        vo[...] = jnp.where(mask, v32, 0.0)

    return pl.pallas_call(
        body,
        grid=(Ep // _BE,),
        in_specs=[
            pl.BlockSpec((_BE, 64), lambda i: (i, 0)),
            pl.BlockSpec((_BE, 32), lambda i: (i, 0)),
            pl.BlockSpec((_BE, 16), lambda i: (i, 0)),
            pl.BlockSpec((8, 32), lambda i: (0, 0)),
            pl.BlockSpec((1, 32), lambda i: (0, 0)),
            pl.BlockSpec((32, 64), lambda i: (0, 0)),
            pl.BlockSpec((1, 64), lambda i: (0, 0)),
        ],
        out_specs=(
            pl.BlockSpec((_BE, 16), lambda i: (i, 0)),
            pl.BlockSpec((_BE, 32), lambda i: (i, 0)),
        ),
        out_shape=(
            jax.ShapeDtypeStruct((Ep, 16), F32),
            jax.ShapeDtypeStruct((Ep, 32), F32),
        ),
    )(K, Q, G, rw1t, rb1, rw2t, rb2)


def _tc_edge2(L, cg, E_real):
    Ep = L.shape[0]

    def body(lr, cr, eo):
        i = pl.program_id(0)
        e8 = jnp.exp(lr[:, 0:8] - cr[:, 0:8])
        cols = lax.broadcasted_iota(I32, (_BE, 8), 1)
        e8 = jnp.where(cols < 4, e8, 0.0)
        rows = i * _BE + lax.broadcasted_iota(I32, (_BE, 1), 0)
        e16 = jnp.concatenate([e8, jnp.zeros((_BE, 8), F32)], axis=1)
        eo[...] = jnp.where(rows < E_real, e16, 0.0)

    return pl.pallas_call(
        body,
        grid=(Ep // _BE,),
        in_specs=[
            pl.BlockSpec((_BE, 16), lambda i: (i, 0)),
            pl.BlockSpec((_BE, 16), lambda i: (i, 0)),
        ],
        out_specs=pl.BlockSpec((_BE, 16), lambda i: (i, 0)),
        out_shape=jax.ShapeDtypeStruct((Ep, 16), F32),
    )(L, cg)


def _tc_edge3(Ee, sg, V, E_real):
    Ep = Ee.shape[0]

    def body(er, sr, vr, mo):
        i = pl.program_id(0)
        alpha = er[:, 0:8] / (sr[:, 0:8] + 1e-9)
        a8 = jnp.concatenate(
            [alpha[:, j // 2:j // 2 + 1] for j in range(8)], axis=1)
        v = vr[...]
        m32 = jnp.concatenate(
            [a8 * v[:, 8 * j:8 * j + 8] for j in range(4)], axis=1)
        rows = i * _BE + lax.broadcasted_iota(I32, (_BE, 1), 0)
        mo[...] = jnp.where(rows < E_real, m32, 0.0)

    return pl.pallas_call(
        body,
        grid=(Ep // _BE,),
        in_specs=[
            pl.BlockSpec((_BE, 16), lambda i: (i, 0)),
            pl.BlockSpec((_BE, 16), lambda i: (i, 0)),
            pl.BlockSpec((_BE, 32), lambda i: (i, 0)),
        ],
        out_specs=pl.BlockSpec((_BE, 32), lambda i: (i, 0)),
        out_shape=jax.ShapeDtypeStruct((Ep, 32), F32),
    )(Ee, sg, V)


def _tc_combine(x2, op):
    _, N8, F = x2.shape

    def body(xr, o):
        if op == "max":
            o[...] = jnp.maximum(xr[0], xr[1])
        else:
            o[...] = xr[0] + xr[1]

    return pl.pallas_call(
        body, out_shape=jax.ShapeDtypeStruct((N8, F), F32)
    )(x2)


def _tc_nodeupd(h0, h1, a2, o0t, o1t, nw, m0t, m1t):
    N8 = h0.shape[0]
    Co = m0t.shape[1]
    n0w1t, n0b1, n0w2t, n0b2, n1w1t, n1b1, n1w2t, n1b2 = nw

    def body(h0r, h1r, ar, o0r, o1r,
             a0w1, a0b1, a0w2, a0b2, a1w1, a1b1, a1w2, a1b2,
             m0r, m1r, h0o, h1o):
        agg = ar[0] + ar[1]
        h0n = h0r[...] + _dot(agg[:, 0:8], o0r[...])
        h1n = [h1r[d] + _dot(agg[:, 8 + 8 * d:16 + 8 * d], o1r[...])
               for d in range(3)]
        n0 = jnp.sqrt(h0n * h0n + 1e-12)
        s0 = _dot(jnp.maximum(_dot(n0, a0w1[...]) + a0b1[...], 0.0),
                  a0w2[...]) + a0b2[...]
        h0g = s0 * h0n / n0
        n1 = jnp.sqrt(h1n[0] * h1n[0] + h1n[1] * h1n[1]
                      + h1n[2] * h1n[2] + 1e-12)
        s1 = _dot(jnp.maximum(_dot(n1, a1w1[...]) + a1b1[...], 0.0),
                  a1w2[...]) + a1b2[...]
        h0o[...] = _dot(h0g, m0r[...])
        for d in range(3):
            h1o[d] = _dot(s1 * h1n[d] / n1, m1r[...])

    return pl.pallas_call(
        body,
        out_shape=(
            jax.ShapeDtypeStruct((N8, Co), F32),
            jax.ShapeDtypeStruct((3, N8, Co), F32),
        ),
    )(h0, h1, a2, o0t, o1t, *nw, m0t, m1t)


def _tc_attend(q, k, v):
    M, Kd = q.shape
    Nc = k.shape[0]
    bM = 600
    scale = 1.0 / math.sqrt(float(_GAS))

    def body(qr, kr, vr, oo, ao):
        logits = lax.dot_general(
            qr[...], kr[...], (((1,), (1,)), ((), ())),
            preferred_element_type=F32) * scale
        m = jnp.max(logits, axis=1, keepdims=True)
        e = jnp.exp(logits - m)
        p = e / jnp.sum(e, axis=1, keepdims=True)
        ao[...] = p
        oo[...] = _dot(p, vr[...])

    return pl.pallas_call(
        body,
        grid=(M // bM,),
        in_specs=[
            pl.BlockSpec((bM, Kd), lambda i: (i, 0)),
            pl.BlockSpec((Nc, Kd), lambda i: (0, 0)),
            pl.BlockSpec((Nc, Kd), lambda i: (0, 0)),
        ],
        out_specs=(
            pl.BlockSpec((bM, Kd), lambda i: (i, 0)),
            pl.BlockSpec((bM, Nc), lambda i: (i, 0)),
        ),
        out_shape=(
            jax.ShapeDtypeStruct((M, Kd), F32),
            jax.ShapeDtypeStruct((M, Nc), F32),
        ),
    )(q, k, v)


def _tc_fedge(Hs, G, fw1t, fb1, fw2t, fb2, E_real):
    Ep = Hs.shape[0]

    def body(hr, gr, w1, bb1, w2, bb2, mo):
        i = pl.program_id(0)
        g = gr[...]
        rf = g[:, 0:8]
        rhat = g[:, 6:9]
        r = jnp.maximum(_dot(rf, w1[...]) + bb1[...], 0.0)
        r = _dot(r, w2[...]) + bb2[...]
        h = hr[...]
        h0s = h[:, 0:32]
        dot3 = sum(h[:, 32 + 32 * d:64 + 32 * d] * rhat[:, d:d + 1]
                   for d in range(3))
        contrib = r[:, 0:32] * h0s + r[:, 32:64] * dot3
        m1 = jnp.sum(contrib, axis=1, keepdims=True)
        m16 = jnp.concatenate([m1, jnp.zeros((_BE, 15), F32)], axis=1)
        rows = i * _BE + lax.broadcasted_iota(I32, (_BE, 1), 0)
        mo[...] = jnp.where(rows < E_real, m16, 0.0)

    return pl.pallas_call(
        body,
        grid=(Ep // _BE,),
        in_specs=[
            pl.BlockSpec((_BE, 128), lambda i: (i, 0)),
            pl.BlockSpec((_BE, 16), lambda i: (i, 0)),
            pl.BlockSpec((8, 32), lambda i: (0, 0)),
            pl.BlockSpec((1, 32), lambda i: (0, 0)),
            pl.BlockSpec((32, 64), lambda i: (0, 0)),
            pl.BlockSpec((1, 64), lambda i: (0, 0)),
        ],
        out_specs=pl.BlockSpec((_BE, 16), lambda i: (i, 0)),
        out_shape=jax.ShapeDtypeStruct((Ep, 16), F32),
    )(Hs, G, fw1t, fb1, fw2t, fb2)


def _tc_finalout(m2, h0, siwt):
    N8 = h0.shape[0]

    def body(mr, hr, sr, o):
        o[...] = mr[0] + mr[1] + _dot(hr[...], sr[...])

    return pl.pallas_call(
        body, out_shape=jax.ShapeDtypeStruct((N8, 16), F32)
    )(m2, h0, siwt)


def _prep_graph(f, pos, ea, ei, p, N, L0):
    E = ei.shape[1]
    N8 = _rup(N + 1, 16)
    Ep = _rup(E, 4096)
    src = jnp.concatenate([ei[0], jnp.full((Ep - E,), N, I32)])
    dst = jnp.concatenate([ei[1], jnp.full((Ep - E,), N, I32)])
    posT = jnp.zeros((N8, 16), F32).at[:N, 0:3].set(pos)
    eap = jnp.zeros((Ep, 5), F32).at[:E].set(ea)
    PS = _sc_gather(posT, src)
    PD = _sc_gather(posT, dst)
    G = _tc_geom(PS, PD, eap)
    fp = jnp.zeros((N8, L0), F32).at[:N].set(f)
    h0 = _tc_init(fp, p['lin1_w'].T, p['lin1_b'][None], p['lin2_w'].T,
                  p['lin2_b'][None])
    h1 = jnp.zeros((3, N8, _C), F32)
    return dict(src=src, dst=dst, G=G, h0=h0, h1=h1, N=N, N8=N8, E=E, Ep=Ep)


def _pad_w1t(w):
    return jnp.zeros((8, w.shape[0]), F32).at[0:6].set(w.T)


def _gse3res_layer(st, lay):
    N8, Ep = st['N8'], st['Ep']
    SRCT, DSTT = _tc_proj(
        st['h0'], st['h1'], lay['q0'].T, lay['q1'].T, lay['k0'].T,
        lay['k1'].T, lay['v0'].T, lay['v1'].T)
    K = _sc_gather(SRCT, st['src'])
    Q = _sc_gather(DSTT, st['dst'])
    L, V = _tc_edge1(K, Q, st['G'], _pad_w1t(lay['rw1']), lay['rb1'][None],
                     lay['rw2'].T, lay['rb2'][None], st['E'])
    c2 = _sc_scatter(L, st['dst'], jnp.full((N8, 16), -1e30, F32), add=False)
    cN = _tc_combine(c2, "max")
    cg = _sc_gather(cN, st['dst'])
    Ee = _tc_edge2(L, cg, st['E'])
    s2 = _sc_scatter(Ee, st['dst'], jnp.zeros((N8, 16), F32), add=True)
    sN = _tc_combine(s2, "add")
    sg = _sc_gather(sN, st['dst'])
    M = _tc_edge3(Ee, sg, V, st['E'])
    a2 = _sc_scatter(M, st['dst'], jnp.zeros((N8, 32), F32), add=True)
    nw = (lay['n0w1'].T, lay['n0b1'][None], lay['n0w2'].T, lay['n0b2'][None],
          lay['n1w1'].T, lay['n1b1'][None], lay['n1w2'].T, lay['n1b2'][None])
    h0n, h1n = _tc_nodeupd(st['h0'], st['h1'], a2, lay['o0'].T, lay['o1'].T,
                           nw, lay['m0'].T, lay['m1'].T)
    st['h0'], st['h1'] = h0n, h1n
    return st


def _final_gconv(st, p):
    N8, N = st['N8'], st['N']
    h0, h1 = st['h0'], st['h1']
    H = jnp.concatenate([h0, h1[0], h1[1], h1[2]], axis=1)
    Hs = _sc_gather(H, st['src'])
    M16 = _tc_fedge(Hs, st['G'], _pad_w1t(p['frw1']), p['frb1'][None],
                    p['frw2'].T, p['frb2'][None], st['E'])
    m2 = _sc_scatter(M16, st['dst'], jnp.zeros((N8, 16), F32), add=True)
    siwt = jnp.zeros((32, 16), F32).at[:, 0].set(p['si_w'][0])
    o16 = _tc_finalout(m2, h0, siwt)
    return o16[:N, 0:1].reshape(N, 1, 1)


def kernel(f_ha, pos_ha, edge_attr_ha, f_ca, pos_ca, edge_attr_ca, params,
           edge_index_ha, edge_index_ca):
    pha, pca = params['ha'], params['ca']
    N_HA, N_CA = f_ha.shape[0], f_ca.shape[0]
    sa = _prep_graph(f_ha, pos_ha, edge_attr_ha, edge_index_ha, pha,
                     N_HA, f_ha.shape[1])
    sc = _prep_graph(f_ca, pos_ca, edge_attr_ca, edge_index_ca, pca,
                     N_CA, f_ca.shape[1])
    attmaps = []
    for i in range(2):
        la, lc = pha['layers'][i], pca['layers'][i]
        sa = _gse3res_layer(sa, la)
        sc = _gse3res_layer(sc, lc)
        h0a, h1a = sa['h0'], sa['h1']
        h0c, h1c = sc['h0'], sc['h1']
        l0m, a0 = _tc_attend(h0a[:N_HA, 0:_GAS], h0c[:N_CA, 0:_GAS],
                             h0c[:N_CA, _GAS:2 * _GAS])
        q1 = h1a[:, :N_HA, 0:_GAS].transpose(1, 2, 0).reshape(N_HA, 3 * _GAS)
        k1 = h1c[:, :N_CA, 0:_GAS].transpose(1, 2, 0).reshape(N_CA, 3 * _GAS)
        v1 = h1c[:, :N_CA, _GAS:2 * _GAS].transpose(1, 2, 0).reshape(
            N_CA, 3 * _GAS)
        l1m, a1 = _tc_attend(q1, k1, v1)
        attmaps += [a0, a1]
        N8a = sa['N8']
        l0p = jnp.zeros((N8a, _GAS), F32).at[:N_HA].set(l0m)
        sa['h0'] = jnp.concatenate([h0a, l0p], axis=1)
        l1r = l1m.reshape(N_HA, _GAS, 3).transpose(2, 0, 1)
        l1p = jnp.zeros((3, N8a, _GAS), F32).at[:, :N_HA].set(l1r)
        sa['h1'] = jnp.concatenate([h1a, l1p], axis=2)
    out_ha = _final_gconv(sa, pha)
    out_ca = _final_gconv(sc, pca)
    return out_ha, out_ca, jnp.stack(attmaps)


# speedup vs baseline: 8.3035x; 2.4994x over previous
import functools
import math

import jax
import jax.numpy as jnp
from jax import lax
from jax.experimental import pallas as pl
from jax.experimental.pallas import tpu as pltpu
from jax.experimental.pallas import tpu_sc as plsc

F32 = jnp.float32
I32 = jnp.int32

_C = 32
_GAS = 16
_CK = 8
_HEADS = 4
_HD = 2
_BE = 2048


def _rup(x, m):
    return (x + m - 1) // m * m


_CH = 128


def _sc_mesh():
    return plsc.VectorSubcoreMesh(core_axis_name="c", subcore_axis_name="s")


def _sc_gather(table, idx):
    N8, F = table.shape
    Ep = idx.shape[0]
    rpt = Ep // 32
    nch = rpt // _CH

    @functools.partial(
        pl.kernel,
        mesh=_sc_mesh(),
        out_type=jax.ShapeDtypeStruct((Ep, F), F32),
        scratch_types=[
            pltpu.VMEM((_CH,), I32),
            pltpu.VMEM((_CH, F), F32),
            pltpu.SemaphoreType.DMA,
        ],
        compiler_params=pltpu.CompilerParams(use_tc_tiling_on_sc=False),
    )
    def k(table_hbm, idx_hbm, out_hbm, idx_v, rows_v, sem):
        wid = lax.axis_index("s") * 2 + lax.axis_index("c")
        base = wid * rpt
        for ci in range(nch):
            off = base + ci * _CH
            pltpu.sync_copy(idx_hbm.at[pl.ds(off, _CH)], idx_v)
            pltpu.async_copy(table_hbm.at[idx_v], rows_v, sem).wait()
            pltpu.sync_copy(rows_v, out_hbm.at[pl.ds(off, _CH)])

    return k(table, idx)


def _sc_scatter(data, idx, init, add):
    Ep, F = data.shape
    N8 = init.shape[0]
    rpt = Ep // 32
    nch = rpt // _CH
    rn = N8 // 16

    @functools.partial(
        pl.kernel,
        mesh=_sc_mesh(),
        out_type=jax.ShapeDtypeStruct((2, N8, F), F32),
        scratch_types=[
            pltpu.VMEM((_CH,), I32),
            pltpu.VMEM((_CH, F), F32),
            pltpu.VMEM_SHARED((N8, F), F32),
        ],
        compiler_params=pltpu.CompilerParams(use_tc_tiling_on_sc=False),
    )
    def k(data_hbm, idx_hbm, init_hbm, out_hbm, idx_v, dat_v, shared):
        cid = lax.axis_index("c")
        sid = lax.axis_index("s")
        wid = sid * 2 + cid
        pltpu.sync_copy(init_hbm.at[pl.ds(sid * rn, rn)],
                        shared.at[pl.ds(sid * rn, rn)])
        plsc.subcore_barrier()
        base = wid * rpt
        for ci in range(nch):
            off = base + ci * _CH
            pltpu.sync_copy(idx_hbm.at[pl.ds(off, _CH)], idx_v)
            pltpu.sync_copy(data_hbm.at[pl.ds(off, _CH)], dat_v)
            pltpu.sync_copy(dat_v, shared.at[idx_v], add=add)
        plsc.subcore_barrier()
        pltpu.sync_copy(shared.at[pl.ds(sid * rn, rn)],
                        out_hbm.at[cid, pl.ds(sid * rn, rn)])

    return k(data, idx, init)


def _dot(a, b):
    return jnp.dot(a, b, preferred_element_type=F32)


def _tc_geom(PS, PD, ea):
    Ep = PS.shape[0]

    def body(ps, pd, a, o):
        rv = pd[:, 0:3] - ps[:, 0:3]
        d = jnp.sqrt(jnp.sum(rv * rv, axis=1, keepdims=True) + 1e-9)
        rhat = rv / d
        z = jnp.zeros((_BE, 7), F32)
        o[...] = jnp.concatenate([a[...], d, rhat, z], axis=1)

    return pl.pallas_call(
        body,
        grid=(Ep // _BE,),
        in_specs=[
            pl.BlockSpec((_BE, 16), lambda i: (i, 0)),
            pl.BlockSpec((_BE, 16), lambda i: (i, 0)),
            pl.BlockSpec((_BE, 5), lambda i: (i, 0)),
        ],
        out_specs=pl.BlockSpec((_BE, 16), lambda i: (i, 0)),
        out_shape=jax.ShapeDtypeStruct((Ep, 16), F32),
    )(PS, PD, ea)


def _tc_init(fp, w1t, b1, w2t, b2):
    N8 = fp.shape[0]

    def body(f, w1, bb1, w2, bb2, o):
        x = _dot(f[...], w1[...]) + bb1[...]
        x = jnp.where(x > 0, x, jnp.exp(x) - 1.0)
        o[...] = _dot(x, w2[...]) + bb2[...]

    return pl.pallas_call(
        body, out_shape=jax.ShapeDtypeStruct((N8, _C), F32)
    )(fp, w1t, b1, w2t, b2)


def _tc_proj(h0, h1, q0t, q1t, k0t, k1t, v0t, v1t):
    N8 = h0.shape[0]

    def body(h0r, h1r, q0r, q1r, k0r, k1r, v0r, v1r, so, do):
        h0v = h0r[...]
        parts_s = [_dot(h0v, k0r[...])]
        for d in range(3):
            parts_s.append(_dot(h1r[d], k1r[...]))
        parts_s.append(_dot(h0v, v0r[...]))
        for d in range(3):
            parts_s.append(_dot(h1r[d], v1r[...]))
        so[...] = jnp.concatenate(parts_s, axis=1)
        parts_d = [_dot(h0v, q0r[...])]
        for d in range(3):
            parts_d.append(_dot(h1r[d], q1r[...]))
        do[...] = jnp.concatenate(parts_d, axis=1)

    return pl.pallas_call(
        body,
        out_shape=(
            jax.ShapeDtypeStruct((N8, 64), F32),
            jax.ShapeDtypeStruct((N8, 32), F32),
        ),
    )(h0, h1, q0t, q1t, k0t, k1t, v0t, v1t)


def _tc_edge1(K, Q, G, rw1t, rb1, rw2t, rb2, E_real):
    Ep = K.shape[0]
    denom = math.sqrt(_HD * 4.0)

    def body(kr, qr, gr, w1, bb1, w2, bb2, lo, vo):
        i = pl.program_id(0)
        g = gr[...]
        rf = g[:, 0:8]
        rhat = g[:, 6:9]
        r = jnp.maximum(_dot(rf, w1[...]) + bb1[...], 0.0)
        r = _dot(r, w2[...]) + bb2[...]
        w = [r[:, 8 * j:8 * j + 8] for j in range(8)]
        kk = kr[...]
        qq = qr[...]
        k0 = kk[:, 0:8]
        v0 = kk[:, 32:40]
        k1 = [kk[:, 8 + 8 * d:16 + 8 * d] for d in range(3)]
        v1 = [kk[:, 40 + 8 * d:48 + 8 * d] for d in range(3)]
        q0 = qq[:, 0:8]
        q1 = [qq[:, 8 + 8 * d:16 + 8 * d] for d in range(3)]
        k1r = sum(k1[d] * rhat[:, d:d + 1] for d in range(3))
        v1r = sum(v1[d] * rhat[:, d:d + 1] for d in range(3))
        key0 = w[0] * k0 + w[2] * k1r
        val0 = w[4] * v0 + w[6] * v1r
        acc = q0 * key0
        for d in range(3):
            key1d = w[3] * k1[d] + (w[1] * k0) * rhat[:, d:d + 1]
            acc = acc + q1[d] * key1d
        l4 = jnp.concatenate(
            [acc[:, 2 * h:2 * h + 1] + acc[:, 2 * h + 1:2 * h + 2]
             for h in range(4)], axis=1) / denom
        rows = i * _BE + lax.broadcasted_iota(I32, (_BE, 1), 0)
        mask = rows < E_real
        l16 = jnp.concatenate([l4, jnp.zeros((_BE, 12), F32)], axis=1)
        lo[...] = jnp.where(mask, l16, 0.0)
        val1s = [w[7] * v1[d] + (w[5] * v0) * rhat[:, d:d + 1] for d in range(3)]
        v32 = jnp.concatenate([val0] + val1s, axis=1)
        vo[...] = jnp.where(mask, v32, 0.0)

    return pl.pallas_call(
        body,
        grid=(Ep // _BE,),
        in_specs=[
            pl.BlockSpec((_BE, 64), lambda i: (i, 0)),
            pl.BlockSpec((_BE, 32), lambda i: (i, 0)),
            pl.BlockSpec((_BE, 16), lambda i: (i, 0)),
            pl.BlockSpec((8, 32), lambda i: (0, 0)),
            pl.BlockSpec((1, 32), lambda i: (0, 0)),
            pl.BlockSpec((32, 64), lambda i: (0, 0)),
            pl.BlockSpec((1, 64), lambda i: (0, 0)),
        ],
        out_specs=(
            pl.BlockSpec((_BE, 16), lambda i: (i, 0)),
            pl.BlockSpec((_BE, 32), lambda i: (i, 0)),
        ),
        out_shape=(
            jax.ShapeDtypeStruct((Ep, 16), F32),
            jax.ShapeDtypeStruct((Ep, 32), F32),
        ),
    )(K, Q, G, rw1t, rb1, rw2t, rb2)


def _tc_edge2(L, cg, E_real):
    Ep = L.shape[0]

    def body(lr, cr, eo):
        i = pl.program_id(0)
        e8 = jnp.exp(lr[:, 0:8] - cr[:, 0:8])
        cols = lax.broadcasted_iota(I32, (_BE, 8), 1)
        e8 = jnp.where(cols < 4, e8, 0.0)
        rows = i * _BE + lax.broadcasted_iota(I32, (_BE, 1), 0)
        e16 = jnp.concatenate([e8, jnp.zeros((_BE, 8), F32)], axis=1)
        eo[...] = jnp.where(rows < E_real, e16, 0.0)

    return pl.pallas_call(
        body,
        grid=(Ep // _BE,),
        in_specs=[
            pl.BlockSpec((_BE, 16), lambda i: (i, 0)),
            pl.BlockSpec((_BE, 16), lambda i: (i, 0)),
        ],
        out_specs=pl.BlockSpec((_BE, 16), lambda i: (i, 0)),
        out_shape=jax.ShapeDtypeStruct((Ep, 16), F32),
    )(L, cg)


def _tc_edge3(Ee, sg, V, E_real):
    Ep = Ee.shape[0]

    def body(er, sr, vr, mo):
        i = pl.program_id(0)
        alpha = er[:, 0:8] / (sr[:, 0:8] + 1e-9)
        a8 = jnp.concatenate(
            [alpha[:, j // 2:j // 2 + 1] for j in range(8)], axis=1)
        v = vr[...]
        m32 = jnp.concatenate(
            [a8 * v[:, 8 * j:8 * j + 8] for j in range(4)], axis=1)
        rows = i * _BE + lax.broadcasted_iota(I32, (_BE, 1), 0)
        mo[...] = jnp.where(rows < E_real, m32, 0.0)

    return pl.pallas_call(
        body,
        grid=(Ep // _BE,),
        in_specs=[
            pl.BlockSpec((_BE, 16), lambda i: (i, 0)),
            pl.BlockSpec((_BE, 16), lambda i: (i, 0)),
            pl.BlockSpec((_BE, 32), lambda i: (i, 0)),
        ],
        out_specs=pl.BlockSpec((_BE, 32), lambda i: (i, 0)),
        out_shape=jax.ShapeDtypeStruct((Ep, 32), F32),
    )(Ee, sg, V)


def _tc_combine(x2, op):
    _, N8, F = x2.shape

    def body(xr, o):
        if op == "max":
            o[...] = jnp.maximum(xr[0], xr[1])
        else:
            o[...] = xr[0] + xr[1]

    return pl.pallas_call(
        body, out_shape=jax.ShapeDtypeStruct((N8, F), F32)
    )(x2)


def _tc_nodeupd(h0, h1, a2, o0t, o1t, nw, m0t, m1t):
    N8 = h0.shape[0]
    Co = m0t.shape[1]
    n0w1t, n0b1, n0w2t, n0b2, n1w1t, n1b1, n1w2t, n1b2 = nw

    def body(h0r, h1r, ar, o0r, o1r,
             a0w1, a0b1, a0w2, a0b2, a1w1, a1b1, a1w2, a1b2,
             m0r, m1r, h0o, h1o):
        agg = ar[0] + ar[1]
        h0n = h0r[...] + _dot(agg[:, 0:8], o0r[...])
        h1n = [h1r[d] + _dot(agg[:, 8 + 8 * d:16 + 8 * d], o1r[...])
               for d in range(3)]
        n0 = jnp.sqrt(h0n * h0n + 1e-12)
        s0 = _dot(jnp.maximum(_dot(n0, a0w1[...]) + a0b1[...], 0.0),
                  a0w2[...]) + a0b2[...]
        h0g = s0 * h0n / n0
        n1 = jnp.sqrt(h1n[0] * h1n[0] + h1n[1] * h1n[1]
                      + h1n[2] * h1n[2] + 1e-12)
        s1 = _dot(jnp.maximum(_dot(n1, a1w1[...]) + a1b1[...], 0.0),
                  a1w2[...]) + a1b2[...]
        h0o[...] = _dot(h0g, m0r[...])
        for d in range(3):
            h1o[d] = _dot(s1 * h1n[d] / n1, m1r[...])

    return pl.pallas_call(
        body,
        out_shape=(
            jax.ShapeDtypeStruct((N8, Co), F32),
            jax.ShapeDtypeStruct((3, N8, Co), F32),
        ),
    )(h0, h1, a2, o0t, o1t, *nw, m0t, m1t)


def _tc_attend(q, k, v):
    M, Kd = q.shape
    Nc = k.shape[0]
    bM = 600
    scale = 1.0 / math.sqrt(float(_GAS))

    def body(qr, kr, vr, oo, ao):
        logits = lax.dot_general(
            qr[...], kr[...], (((1,), (1,)), ((), ())),
            preferred_element_type=F32) * scale
        m = jnp.max(logits, axis=1, keepdims=True)
        e = jnp.exp(logits - m)
        p = e / jnp.sum(e, axis=1, keepdims=True)
        ao[...] = p
        oo[...] = _dot(p, vr[...])

    return pl.pallas_call(
        body,
        grid=(M // bM,),
        in_specs=[
            pl.BlockSpec((bM, Kd), lambda i: (i, 0)),
            pl.BlockSpec((Nc, Kd), lambda i: (0, 0)),
            pl.BlockSpec((Nc, Kd), lambda i: (0, 0)),
        ],
        out_specs=(
            pl.BlockSpec((bM, Kd), lambda i: (i, 0)),
            pl.BlockSpec((bM, Nc), lambda i: (i, 0)),
        ),
        out_shape=(
            jax.ShapeDtypeStruct((M, Kd), F32),
            jax.ShapeDtypeStruct((M, Nc), F32),
        ),
    )(q, k, v)


def _tc_fedge(Hs, G, fw1t, fb1, fw2t, fb2, E_real):
    Ep = Hs.shape[0]

    def body(hr, gr, w1, bb1, w2, bb2, mo):
        i = pl.program_id(0)
        g = gr[...]
        rf = g[:, 0:8]
        rhat = g[:, 6:9]
        r = jnp.maximum(_dot(rf, w1[...]) + bb1[...], 0.0)
        r = _dot(r, w2[...]) + bb2[...]
        h = hr[...]
        h0s = h[:, 0:32]
        dot3 = sum(h[:, 32 + 32 * d:64 + 32 * d] * rhat[:, d:d + 1]
                   for d in range(3))
        contrib = r[:, 0:32] * h0s + r[:, 32:64] * dot3
        m1 = jnp.sum(contrib, axis=1, keepdims=True)
        m16 = jnp.concatenate([m1, jnp.zeros((_BE, 15), F32)], axis=1)
        rows = i * _BE + lax.broadcasted_iota(I32, (_BE, 1), 0)
        mo[...] = jnp.where(rows < E_real, m16, 0.0)

    return pl.pallas_call(
        body,
        grid=(Ep // _BE,),
        in_specs=[
            pl.BlockSpec((_BE, 128), lambda i: (i, 0)),
            pl.BlockSpec((_BE, 16), lambda i: (i, 0)),
            pl.BlockSpec((8, 32), lambda i: (0, 0)),
            pl.BlockSpec((1, 32), lambda i: (0, 0)),
            pl.BlockSpec((32, 64), lambda i: (0, 0)),
            pl.BlockSpec((1, 64), lambda i: (0, 0)),
        ],
        out_specs=pl.BlockSpec((_BE, 16), lambda i: (i, 0)),
        out_shape=jax.ShapeDtypeStruct((Ep, 16), F32),
    )(Hs, G, fw1t, fb1, fw2t, fb2)


def _tc_finalout(m2, h0, siwt):
    N8 = h0.shape[0]

    def body(mr, hr, sr, o):
        o[...] = mr[0] + mr[1] + _dot(hr[...], sr[...])

    return pl.pallas_call(
        body, out_shape=jax.ShapeDtypeStruct((N8, 16), F32)
    )(m2, h0, siwt)


def _prep_graph(f, pos, ea, ei, p, N, L0):
    E = ei.shape[1]
    N8 = _rup(N + 1, 16)
    Ep = _rup(E, 4096)
    src = jnp.concatenate([ei[0], jnp.full((Ep - E,), N, I32)])
    dst = jnp.concatenate([ei[1], jnp.full((Ep - E,), N, I32)])
    posT = jnp.zeros((N8, 16), F32).at[:N, 0:3].set(pos)
    eap = jnp.zeros((Ep, 5), F32).at[:E].set(ea)
    PS = _sc_gather(posT, src)
    PD = _sc_gather(posT, dst)
    G = _tc_geom(PS, PD, eap)
    fp = jnp.zeros((N8, L0), F32).at[:N].set(f)
    h0 = _tc_init(fp, p['lin1_w'].T, p['lin1_b'][None], p['lin2_w'].T,
                  p['lin2_b'][None])
    h1 = jnp.zeros((3, N8, _C), F32)
    return dict(src=src, dst=dst, G=G, h0=h0, h1=h1, N=N, N8=N8, E=E, Ep=Ep)


def _pad_w1t(w):
    return jnp.zeros((8, w.shape[0]), F32).at[0:6].set(w.T)


def _gse3res_layer(st, lay):
    N8, Ep = st['N8'], st['Ep']
    SRCT, DSTT = _tc_proj(
        st['h0'], st['h1'], lay['q0'].T, lay['q1'].T, lay['k0'].T,
        lay['k1'].T, lay['v0'].T, lay['v1'].T)
    K = _sc_gather(SRCT, st['src'])
    Q = _sc_gather(DSTT, st['dst'])
    L, V = _tc_edge1(K, Q, st['G'], _pad_w1t(lay['rw1']), lay['rb1'][None],
                     lay['rw2'].T, lay['rb2'][None], st['E'])
    c2 = _sc_scatter(L, st['dst'], jnp.full((N8, 16), -1e30, F32), add=False)
    cN = _tc_combine(c2, "max")
    cg = _sc_gather(cN, st['dst'])
    Ee = _tc_edge2(L, cg, st['E'])
    s2 = _sc_scatter(Ee, st['dst'], jnp.zeros((N8, 16), F32), add=True)
    sN = _tc_combine(s2, "add")
    sg = _sc_gather(sN, st['dst'])
    M = _tc_edge3(Ee, sg, V, st['E'])
    a2 = _sc_scatter(M, st['dst'], jnp.zeros((N8, 32), F32), add=True)
    nw = (lay['n0w1'].T, lay['n0b1'][None], lay['n0w2'].T, lay['n0b2'][None],
          lay['n1w1'].T, lay['n1b1'][None], lay['n1w2'].T, lay['n1b2'][None])
    h0n, h1n = _tc_nodeupd(st['h0'], st['h1'], a2, lay['o0'].T, lay['o1'].T,
                           nw, lay['m0'].T, lay['m1'].T)
    st['h0'], st['h1'] = h0n, h1n
    return st


def _final_gconv(st, p):
    N8, N = st['N8'], st['N']
    h0, h1 = st['h0'], st['h1']
    H = jnp.concatenate([h0, h1[0], h1[1], h1[2]], axis=1)
    Hs = _sc_gather(H, st['src'])
    M16 = _tc_fedge(Hs, st['G'], _pad_w1t(p['frw1']), p['frb1'][None],
                    p['frw2'].T, p['frb2'][None], st['E'])
    m2 = _sc_scatter(M16, st['dst'], jnp.zeros((N8, 16), F32), add=True)
    siwt = jnp.zeros((32, 16), F32).at[:, 0].set(p['si_w'][0])
    o16 = _tc_finalout(m2, h0, siwt)
    return o16[:N, 0:1].reshape(N, 1, 1)


def kernel(f_ha, pos_ha, edge_attr_ha, f_ca, pos_ca, edge_attr_ca, params,
           edge_index_ha, edge_index_ca):
    pha, pca = params['ha'], params['ca']
    N_HA, N_CA = f_ha.shape[0], f_ca.shape[0]
    sa = _prep_graph(f_ha, pos_ha, edge_attr_ha, edge_index_ha, pha,
                     N_HA, f_ha.shape[1])
    sc = _prep_graph(f_ca, pos_ca, edge_attr_ca, edge_index_ca, pca,
                     N_CA, f_ca.shape[1])
    attmaps = []
    for i in range(2):
        la, lc = pha['layers'][i], pca['layers'][i]
        sa = _gse3res_layer(sa, la)
        sc = _gse3res_layer(sc, lc)
        h0a, h1a = sa['h0'], sa['h1']
        h0c, h1c = sc['h0'], sc['h1']
        l0m, a0 = _tc_attend(h0a[:N_HA, 0:_GAS], h0c[:N_CA, 0:_GAS],
                             h0c[:N_CA, _GAS:2 * _GAS])
        q1 = h1a[:, :N_HA, 0:_GAS].transpose(1, 2, 0).reshape(N_HA, 3 * _GAS)
        k1 = h1c[:, :N_CA, 0:_GAS].transpose(1, 2, 0).reshape(N_CA, 3 * _GAS)
        v1 = h1c[:, :N_CA, _GAS:2 * _GAS].transpose(1, 2, 0).reshape(
            N_CA, 3 * _GAS)
        l1m, a1 = _tc_attend(q1, k1, v1)
        attmaps += [a0, a1]
        N8a = sa['N8']
        l0p = jnp.zeros((N8a, _GAS), F32).at[:N_HA].set(l0m)
        sa['h0'] = jnp.concatenate([h0a, l0p], axis=1)
        l1r = l1m.reshape(N_HA, _GAS, 3).transpose(2, 0, 1)
        l1p = jnp.zeros((3, N8a, _GAS), F32).at[:, :N_HA].set(l1r)
        sa['h1'] = jnp.concatenate([h1a, l1p], axis=2)
    out_ha = _final_gconv(sa, pha)
    out_ca = _final_gconv(sc, pca)
    return out_ha, out_ca, jnp.stack(attmaps)


# speedup vs baseline: 8.9995x; 1.0838x over previous
import functools
import math

import jax
import jax.numpy as jnp
from jax import lax
from jax.experimental import pallas as pl
from jax.experimental.pallas import tpu as pltpu
from jax.experimental.pallas import tpu_sc as plsc

F32 = jnp.float32
I32 = jnp.int32

_C = 32
_GAS = 16
_CK = 8
_HEADS = 4
_HD = 2
_BE = 2048


def _rup(x, m):
    return (x + m - 1) // m * m


_CH = 128


def _sc_mesh():
    return plsc.VectorSubcoreMesh(core_axis_name="c", subcore_axis_name="s")


def _sc_gather(table, idx):
    N8, F = table.shape
    Ep = idx.shape[0]
    rpt = Ep // 32
    nch = rpt // _CH

    @functools.partial(
        pl.kernel,
        mesh=_sc_mesh(),
        out_type=jax.ShapeDtypeStruct((Ep, F), F32),
        scratch_types=[
            pltpu.VMEM((_CH,), I32),
            pltpu.VMEM((_CH, F), F32),
            pltpu.SemaphoreType.DMA,
        ],
        compiler_params=pltpu.CompilerParams(use_tc_tiling_on_sc=False),
    )
    def k(table_hbm, idx_hbm, out_hbm, idx_v, rows_v, sem):
        wid = lax.axis_index("s") * 2 + lax.axis_index("c")
        base = wid * rpt
        for ci in range(nch):
            off = base + ci * _CH
            pltpu.sync_copy(idx_hbm.at[pl.ds(off, _CH)], idx_v)
            pltpu.async_copy(table_hbm.at[idx_v], rows_v, sem).wait()
            pltpu.sync_copy(rows_v, out_hbm.at[pl.ds(off, _CH)])

    return k(table, idx)


def _sc_scatter(data, idx, init, add):
    Ep, F = data.shape
    N8 = init.shape[0]
    rpt = Ep // 32
    nch = rpt // _CH
    rn = N8 // 16

    @functools.partial(
        pl.kernel,
        mesh=_sc_mesh(),
        out_type=jax.ShapeDtypeStruct((2, N8, F), F32),
        scratch_types=[
            pltpu.VMEM((_CH,), I32),
            pltpu.VMEM((_CH, F), F32),
            pltpu.VMEM_SHARED((N8, F), F32),
        ],
        compiler_params=pltpu.CompilerParams(use_tc_tiling_on_sc=False),
    )
    def k(data_hbm, idx_hbm, init_hbm, out_hbm, idx_v, dat_v, shared):
        cid = lax.axis_index("c")
        sid = lax.axis_index("s")
        wid = sid * 2 + cid
        pltpu.sync_copy(init_hbm.at[pl.ds(sid * rn, rn)],
                        shared.at[pl.ds(sid * rn, rn)])
        plsc.subcore_barrier()
        base = wid * rpt
        for ci in range(nch):
            off = base + ci * _CH
            pltpu.sync_copy(idx_hbm.at[pl.ds(off, _CH)], idx_v)
            pltpu.sync_copy(data_hbm.at[pl.ds(off, _CH)], dat_v)
            pltpu.sync_copy(dat_v, shared.at[idx_v], add=add)
        plsc.subcore_barrier()
        pltpu.sync_copy(shared.at[pl.ds(sid * rn, rn)],
                        out_hbm.at[cid, pl.ds(sid * rn, rn)])

    return k(data, idx, init)


def _dot(a, b):
    return jnp.dot(a, b, preferred_element_type=F32)


def _tc_geom(PS, PD, ea):
    Ep = PS.shape[0]

    def body(ps, pd, a, o):
        rv = pd[:, 0:3] - ps[:, 0:3]
        d = jnp.sqrt(jnp.sum(rv * rv, axis=1, keepdims=True) + 1e-9)
        rhat = rv / d
        z = jnp.zeros((_BE, 7), F32)
        o[...] = jnp.concatenate([a[...], d, rhat, z], axis=1)

    return pl.pallas_call(
        body,
        grid=(Ep // _BE,),
        in_specs=[
            pl.BlockSpec((_BE, 16), lambda i: (i, 0)),
            pl.BlockSpec((_BE, 16), lambda i: (i, 0)),
            pl.BlockSpec((_BE, 5), lambda i: (i, 0)),
        ],
        out_specs=pl.BlockSpec((_BE, 16), lambda i: (i, 0)),
        out_shape=jax.ShapeDtypeStruct((Ep, 16), F32),
    )(PS, PD, ea)


def _tc_init(fp, w1t, b1, w2t, b2):
    N8 = fp.shape[0]

    def body(f, w1, bb1, w2, bb2, o):
        x = _dot(f[...], w1[...]) + bb1[...]
        x = jnp.where(x > 0, x, jnp.exp(x) - 1.0)
        o[...] = _dot(x, w2[...]) + bb2[...]

    return pl.pallas_call(
        body, out_shape=jax.ShapeDtypeStruct((N8, _C), F32)
    )(fp, w1t, b1, w2t, b2)


def _tc_proj(h0, h1, q0t, q1t, k0t, k1t, v0t, v1t):
    N8 = h0.shape[0]

    def body(h0r, h1r, q0r, q1r, k0r, k1r, v0r, v1r, so, do):
        h0v = h0r[...]
        parts_s = [_dot(h0v, k0r[...])]
        for d in range(3):
            parts_s.append(_dot(h1r[d], k1r[...]))
        parts_s.append(_dot(h0v, v0r[...]))
        for d in range(3):
            parts_s.append(_dot(h1r[d], v1r[...]))
        so[...] = jnp.concatenate(parts_s, axis=1)
        parts_d = [_dot(h0v, q0r[...])]
        for d in range(3):
            parts_d.append(_dot(h1r[d], q1r[...]))
        do[...] = jnp.concatenate(parts_d, axis=1)

    return pl.pallas_call(
        body,
        out_shape=(
            jax.ShapeDtypeStruct((N8, 64), F32),
            jax.ShapeDtypeStruct((N8, 32), F32),
        ),
    )(h0, h1, q0t, q1t, k0t, k1t, v0t, v1t)


def _tc_edge1(K, Q, G, rw1t, rb1, rw2t, rb2, E_real):
    Ep = K.shape[0]
    denom = math.sqrt(_HD * 4.0)

    def body(kr, qr, gr, w1, bb1, w2, bb2, lo, vo):
        i = pl.program_id(0)
        g = gr[...]
        rf = g[:, 0:8]
        rhat = g[:, 6:9]
        r = jnp.maximum(_dot(rf, w1[...]) + bb1[...], 0.0)
        r = _dot(r, w2[...]) + bb2[...]
        w = [r[:, 8 * j:8 * j + 8] for j in range(8)]
        kk = kr[...]
        qq = qr[...]
        k0 = kk[:, 0:8]
        v0 = kk[:, 32:40]
        k1 = [kk[:, 8 + 8 * d:16 + 8 * d] for d in range(3)]
        v1 = [kk[:, 40 + 8 * d:48 + 8 * d] for d in range(3)]
        q0 = qq[:, 0:8]
        q1 = [qq[:, 8 + 8 * d:16 + 8 * d] for d in range(3)]
        k1r = sum(k1[d] * rhat[:, d:d + 1] for d in range(3))
        v1r = sum(v1[d] * rhat[:, d:d + 1] for d in range(3))
        key0 = w[0] * k0 + w[2] * k1r
        val0 = w[4] * v0 + w[6] * v1r
        acc = q0 * key0
        for d in range(3):
            key1d = w[3] * k1[d] + (w[1] * k0) * rhat[:, d:d + 1]
            acc = acc + q1[d] * key1d
        l4 = jnp.concatenate(
            [acc[:, 2 * h:2 * h + 1] + acc[:, 2 * h + 1:2 * h + 2]
             for h in range(4)], axis=1) / denom
        rows = i * _BE + lax.broadcasted_iota(I32, (_BE, 1), 0)
        mask = rows < E_real
        l16 = jnp.concatenate([l4, jnp.zeros((_BE, 12), F32)], axis=1)
        lo[...] = jnp.where(mask, l16, 0.0)
        val1s = [w[7] * v1[d] + (w[5] * v0) * rhat[:, d:d + 1] for d in range(3)]
        v32 = jnp.concatenate([val0] + val1s, axis=1)
        vo[...] = jnp.where(mask, v32, 0.0)

    return pl.pallas_call(
        body,
        grid=(Ep // _BE,),
        in_specs=[
            pl.BlockSpec((_BE, 64), lambda i: (i, 0)),
            pl.BlockSpec((_BE, 32), lambda i: (i, 0)),
            pl.BlockSpec((_BE, 16), lambda i: (i, 0)),
            pl.BlockSpec((8, 32), lambda i: (0, 0)),
            pl.BlockSpec((1, 32), lambda i: (0, 0)),
            pl.BlockSpec((32, 64), lambda i: (0, 0)),
            pl.BlockSpec((1, 64), lambda i: (0, 0)),
        ],
        out_specs=(
            pl.BlockSpec((_BE, 16), lambda i: (i, 0)),
            pl.BlockSpec((_BE, 32), lambda i: (i, 0)),
        ),
        out_shape=(
            jax.ShapeDtypeStruct((Ep, 16), F32),
            jax.ShapeDtypeStruct((Ep, 32), F32),
        ),
    )(K, Q, G, rw1t, rb1, rw2t, rb2)


def _tc_edge2(L, cg, E_real):
    Ep = L.shape[0]

    def body(lr, cr, eo):
        i = pl.program_id(0)
        e8 = jnp.exp(lr[:, 0:8] - cr[:, 0:8])
        cols = lax.broadcasted_iota(I32, (_BE, 8), 1)
        e8 = jnp.where(cols < 4, e8, 0.0)
        rows = i * _BE + lax.broadcasted_iota(I32, (_BE, 1), 0)
        e16 = jnp.concatenate([e8, jnp.zeros((_BE, 8), F32)], axis=1)
        eo[...] = jnp.where(rows < E_real, e16, 0.0)

    return pl.pallas_call(
        body,
        grid=(Ep // _BE,),
        in_specs=[
            pl.BlockSpec((_BE, 16), lambda i: (i, 0)),
            pl.BlockSpec((_BE, 16), lambda i: (i, 0)),
        ],
        out_specs=pl.BlockSpec((_BE, 16), lambda i: (i, 0)),
        out_shape=jax.ShapeDtypeStruct((Ep, 16), F32),
    )(L, cg)


def _tc_edge3(Ee, sg, V, E_real):
    Ep = Ee.shape[0]

    def body(er, sr, vr, mo):
        i = pl.program_id(0)
        alpha = er[:, 0:8] / (sr[:, 0:8] + 1e-9)
        a8 = jnp.concatenate(
            [alpha[:, j // 2:j // 2 + 1] for j in range(8)], axis=1)
        v = vr[...]
        m32 = jnp.concatenate(
            [a8 * v[:, 8 * j:8 * j + 8] for j in range(4)], axis=1)
        rows = i * _BE + lax.broadcasted_iota(I32, (_BE, 1), 0)
        mo[...] = jnp.where(rows < E_real, m32, 0.0)

    return pl.pallas_call(
        body,
        grid=(Ep // _BE,),
        in_specs=[
            pl.BlockSpec((_BE, 16), lambda i: (i, 0)),
            pl.BlockSpec((_BE, 16), lambda i: (i, 0)),
            pl.BlockSpec((_BE, 32), lambda i: (i, 0)),
        ],
        out_specs=pl.BlockSpec((_BE, 32), lambda i: (i, 0)),
        out_shape=jax.ShapeDtypeStruct((Ep, 32), F32),
    )(Ee, sg, V)


def _tc_combine(x2, op):
    _, N8, F = x2.shape

    def body(xr, o):
        if op == "max":
            o[...] = jnp.maximum(xr[0], xr[1])
        else:
            o[...] = xr[0] + xr[1]

    return pl.pallas_call(
        body, out_shape=jax.ShapeDtypeStruct((N8, F), F32)
    )(x2)


def _tc_nodeupd(h0, h1, a2, o0t, o1t, nw, m0t, m1t):
    N8 = h0.shape[0]
    Co = m0t.shape[1]
    n0w1t, n0b1, n0w2t, n0b2, n1w1t, n1b1, n1w2t, n1b2 = nw

    def body(h0r, h1r, ar, o0r, o1r,
             a0w1, a0b1, a0w2, a0b2, a1w1, a1b1, a1w2, a1b2,
             m0r, m1r, h0o, h1o):
        agg = ar[0] + ar[1]
        h0n = h0r[...] + _dot(agg[:, 0:8], o0r[...])
        h1n = [h1r[d] + _dot(agg[:, 8 + 8 * d:16 + 8 * d], o1r[...])
               for d in range(3)]
        n0 = jnp.sqrt(h0n * h0n + 1e-12)
        s0 = _dot(jnp.maximum(_dot(n0, a0w1[...]) + a0b1[...], 0.0),
                  a0w2[...]) + a0b2[...]
        h0g = s0 * h0n / n0
        n1 = jnp.sqrt(h1n[0] * h1n[0] + h1n[1] * h1n[1]
                      + h1n[2] * h1n[2] + 1e-12)
        s1 = _dot(jnp.maximum(_dot(n1, a1w1[...]) + a1b1[...], 0.0),
                  a1w2[...]) + a1b2[...]
        h0o[...] = _dot(h0g, m0r[...])
        for d in range(3):
            h1o[d] = _dot(s1 * h1n[d] / n1, m1r[...])

    return pl.pallas_call(
        body,
        out_shape=(
            jax.ShapeDtypeStruct((N8, Co), F32),
            jax.ShapeDtypeStruct((3, N8, Co), F32),
        ),
    )(h0, h1, a2, o0t, o1t, *nw, m0t, m1t)


def _tc_attend(q, k, v):
    M, Kd = q.shape
    Nc = k.shape[0]
    bM = 600
    scale = 1.0 / math.sqrt(float(_GAS))

    def body(qr, kr, vr, oo, ao):
        logits = lax.dot_general(
            qr[...], kr[...], (((1,), (1,)), ((), ())),
            preferred_element_type=F32) * scale
        m = jnp.max(logits, axis=1, keepdims=True)
        e = jnp.exp(logits - m)
        p = e / jnp.sum(e, axis=1, keepdims=True)
        ao[...] = p
        oo[...] = _dot(p, vr[...])

    return pl.pallas_call(
        body,
        grid=(M // bM,),
        in_specs=[
            pl.BlockSpec((bM, Kd), lambda i: (i, 0)),
            pl.BlockSpec((Nc, Kd), lambda i: (0, 0)),
            pl.BlockSpec((Nc, Kd), lambda i: (0, 0)),
        ],
        out_specs=(
            pl.BlockSpec((bM, Kd), lambda i: (i, 0)),
            pl.BlockSpec((bM, Nc), lambda i: (i, 0)),
        ),
        out_shape=(
            jax.ShapeDtypeStruct((M, Kd), F32),
            jax.ShapeDtypeStruct((M, Nc), F32),
        ),
    )(q, k, v)


def _tc_stack4(a, b, c, d):
    M, Nc = a.shape
    bM = 600

    def body(ar, br, cr, dr, o):
        o[0] = ar[...]
        o[1] = br[...]
        o[2] = cr[...]
        o[3] = dr[...]

    return pl.pallas_call(
        body,
        grid=(M // bM,),
        in_specs=[pl.BlockSpec((bM, Nc), lambda i: (i, 0))] * 4,
        out_specs=pl.BlockSpec((4, bM, Nc), lambda i: (0, i, 0)),
        out_shape=jax.ShapeDtypeStruct((4, M, Nc), F32),
    )(a, b, c, d)


def _tc_fedge(Hs, G, fw1t, fb1, fw2t, fb2, E_real):
    Ep = Hs.shape[0]

    def body(hr, gr, w1, bb1, w2, bb2, mo):
        i = pl.program_id(0)
        g = gr[...]
        rf = g[:, 0:8]
        rhat = g[:, 6:9]
        r = jnp.maximum(_dot(rf, w1[...]) + bb1[...], 0.0)
        r = _dot(r, w2[...]) + bb2[...]
        h = hr[...]
        h0s = h[:, 0:32]
        dot3 = sum(h[:, 32 + 32 * d:64 + 32 * d] * rhat[:, d:d + 1]
                   for d in range(3))
        contrib = r[:, 0:32] * h0s + r[:, 32:64] * dot3
        m1 = jnp.sum(contrib, axis=1, keepdims=True)
        m16 = jnp.concatenate([m1, jnp.zeros((_BE, 15), F32)], axis=1)
        rows = i * _BE + lax.broadcasted_iota(I32, (_BE, 1), 0)
        mo[...] = jnp.where(rows < E_real, m16, 0.0)

    return pl.pallas_call(
        body,
        grid=(Ep // _BE,),
        in_specs=[
            pl.BlockSpec((_BE, 128), lambda i: (i, 0)),
            pl.BlockSpec((_BE, 16), lambda i: (i, 0)),
            pl.BlockSpec((8, 32), lambda i: (0, 0)),
            pl.BlockSpec((1, 32), lambda i: (0, 0)),
            pl.BlockSpec((32, 64), lambda i: (0, 0)),
            pl.BlockSpec((1, 64), lambda i: (0, 0)),
        ],
        out_specs=pl.BlockSpec((_BE, 16), lambda i: (i, 0)),
        out_shape=jax.ShapeDtypeStruct((Ep, 16), F32),
    )(Hs, G, fw1t, fb1, fw2t, fb2)


def _tc_finalout(m2, h0, siwt):
    N8 = h0.shape[0]

    def body(mr, hr, sr, o):
        o[...] = mr[0] + mr[1] + _dot(hr[...], sr[...])

    return pl.pallas_call(
        body, out_shape=jax.ShapeDtypeStruct((N8, 16), F32)
    )(m2, h0, siwt)


def _prep_graph(f, pos, ea, ei, p, N, L0):
    E = ei.shape[1]
    N8 = _rup(N + 1, 16)
    Ep = _rup(E, 4096)
    src = jnp.concatenate([ei[0], jnp.full((Ep - E,), N, I32)])
    dst = jnp.concatenate([ei[1], jnp.full((Ep - E,), N, I32)])
    posT = jnp.zeros((N8, 16), F32).at[:N, 0:3].set(pos)
    eap = jnp.zeros((Ep, 5), F32).at[:E].set(ea)
    PS = _sc_gather(posT, src)
    PD = _sc_gather(posT, dst)
    G = _tc_geom(PS, PD, eap)
    fp = jnp.zeros((N8, L0), F32).at[:N].set(f)
    h0 = _tc_init(fp, p['lin1_w'].T, p['lin1_b'][None], p['lin2_w'].T,
                  p['lin2_b'][None])
    h1 = jnp.zeros((3, N8, _C), F32)
    return dict(src=src, dst=dst, G=G, h0=h0, h1=h1, N=N, N8=N8, E=E, Ep=Ep)


def _pad_w1t(w):
    return jnp.zeros((8, w.shape[0]), F32).at[0:6].set(w.T)


def _gse3res_layer(st, lay):
    N8, Ep = st['N8'], st['Ep']
    SRCT, DSTT = _tc_proj(
        st['h0'], st['h1'], lay['q0'].T, lay['q1'].T, lay['k0'].T,
        lay['k1'].T, lay['v0'].T, lay['v1'].T)
    K = _sc_gather(SRCT, st['src'])
    Q = _sc_gather(DSTT, st['dst'])
    L, V = _tc_edge1(K, Q, st['G'], _pad_w1t(lay['rw1']), lay['rb1'][None],
                     lay['rw2'].T, lay['rb2'][None], st['E'])
    c2 = _sc_scatter(L, st['dst'], jnp.full((N8, 16), -1e30, F32), add=False)
    cN = _tc_combine(c2, "max")
    cg = _sc_gather(cN, st['dst'])
    Ee = _tc_edge2(L, cg, st['E'])
    s2 = _sc_scatter(Ee, st['dst'], jnp.zeros((N8, 16), F32), add=True)
    sN = _tc_combine(s2, "add")
    sg = _sc_gather(sN, st['dst'])
    M = _tc_edge3(Ee, sg, V, st['E'])
    a2 = _sc_scatter(M, st['dst'], jnp.zeros((N8, 32), F32), add=True)
    nw = (lay['n0w1'].T, lay['n0b1'][None], lay['n0w2'].T, lay['n0b2'][None],
          lay['n1w1'].T, lay['n1b1'][None], lay['n1w2'].T, lay['n1b2'][None])
    h0n, h1n = _tc_nodeupd(st['h0'], st['h1'], a2, lay['o0'].T, lay['o1'].T,
                           nw, lay['m0'].T, lay['m1'].T)
    st['h0'], st['h1'] = h0n, h1n
    return st


def _final_gconv(st, p):
    N8, N = st['N8'], st['N']
    h0, h1 = st['h0'], st['h1']
    H = jnp.concatenate([h0, h1[0], h1[1], h1[2]], axis=1)
    Hs = _sc_gather(H, st['src'])
    M16 = _tc_fedge(Hs, st['G'], _pad_w1t(p['frw1']), p['frb1'][None],
                    p['frw2'].T, p['frb2'][None], st['E'])
    m2 = _sc_scatter(M16, st['dst'], jnp.zeros((N8, 16), F32), add=True)
    siwt = jnp.zeros((32, 16), F32).at[:, 0].set(p['si_w'][0])
    o16 = _tc_finalout(m2, h0, siwt)
    return o16[:N, 0:1].reshape(N, 1, 1)


def kernel(f_ha, pos_ha, edge_attr_ha, f_ca, pos_ca, edge_attr_ca, params,
           edge_index_ha, edge_index_ca):
    pha, pca = params['ha'], params['ca']
    N_HA, N_CA = f_ha.shape[0], f_ca.shape[0]
    sa = _prep_graph(f_ha, pos_ha, edge_attr_ha, edge_index_ha, pha,
                     N_HA, f_ha.shape[1])
    sc = _prep_graph(f_ca, pos_ca, edge_attr_ca, edge_index_ca, pca,
                     N_CA, f_ca.shape[1])
    attmaps = []
    for i in range(2):
        la, lc = pha['layers'][i], pca['layers'][i]
        sa = _gse3res_layer(sa, la)
        sc = _gse3res_layer(sc, lc)
        h0a, h1a = sa['h0'], sa['h1']
        h0c, h1c = sc['h0'], sc['h1']
        l0m, a0 = _tc_attend(h0a[:N_HA, 0:_GAS], h0c[:N_CA, 0:_GAS],
                             h0c[:N_CA, _GAS:2 * _GAS])
        q1 = h1a[:, :N_HA, 0:_GAS].transpose(1, 2, 0).reshape(N_HA, 3 * _GAS)
        k1 = h1c[:, :N_CA, 0:_GAS].transpose(1, 2, 0).reshape(N_CA, 3 * _GAS)
        v1 = h1c[:, :N_CA, _GAS:2 * _GAS].transpose(1, 2, 0).reshape(
            N_CA, 3 * _GAS)
        l1m, a1 = _tc_attend(q1, k1, v1)
        attmaps += [a0, a1]
        N8a = sa['N8']
        l0p = jnp.zeros((N8a, _GAS), F32).at[:N_HA].set(l0m)
        sa['h0'] = jnp.concatenate([h0a, l0p], axis=1)
        l1r = l1m.reshape(N_HA, _GAS, 3).transpose(2, 0, 1)
        l1p = jnp.zeros((3, N8a, _GAS), F32).at[:, :N_HA].set(l1r)
        sa['h1'] = jnp.concatenate([h1a, l1p], axis=2)
    out_ha = _final_gconv(sa, pha)
    out_ca = _final_gconv(sc, pca)
    return out_ha, out_ca, _tc_stack4(*attmaps)
